# Initial kernel scaffold; baseline (speedup 1.0000x reference)
import jax
import jax.numpy as jnp
from jax.experimental import pallas as pl


def kernel(x, codebook, scale):
    raise NotImplementedError("write your pallas kernel here")


# speedup vs baseline: 2.0751x; 2.0751x over previous
import jax
import jax.numpy as jnp
from jax.experimental import pallas as pl
from jax.experimental.pallas import tpu as pltpu

_M = 4
_K = 1024
_D = 64
_N = 16384
_BLK = 512


def _drq_block_kernel(scale_ref, x_ref, cb_ref, cbt_ref, codes_ref, loss_ref):
    pid = pl.program_id(0)

    c = cb_ref[...]
    ct = cbt_ref[...]
    x0 = x_ref[...]
    r = x0
    qsoft = jnp.zeros_like(x0)
    qhard = jnp.zeros_like(x0)
    col_ids = jax.lax.broadcasted_iota(jnp.int32, (_BLK, _K), 1)

    sd = jnp.float32(0.0)
    hd = jnp.float32(0.0)
    codes_cols = []
    for i in range(_M):
        s = scale_ref[i]
        ci = c * s
        cit = ct * s
        cn = jnp.sum(ci * ci, axis=1)
        rn = jnp.sum(r * r, axis=1, keepdims=True)
        dots = jax.lax.dot_general(
            r, cit, (((1,), (0,)), ((), ())),
            preferred_element_type=jnp.float32)
        logits = -((rn - 2.0 * dots) + cn[None, :])
        mx = jnp.max(logits, axis=1, keepdims=True)
        e = jnp.exp(logits - mx)
        z = jnp.sum(e, axis=1, keepdims=True)
        soft = jax.lax.dot_general(
            e / z, ci, (((1,), (0,)), ((), ())),
            preferred_element_type=jnp.float32)
        code = jnp.min(jnp.where(logits == mx, col_ids, _K),
                       axis=1, keepdims=True)
        onehot = (col_ids == code).astype(jnp.float32)
        hard = jax.lax.dot_general(
            onehot, ci, (((1,), (0,)), ((), ())),
            preferred_element_type=jnp.float32)
        r = r - hard
        qsoft = qsoft + soft
        qhard = qhard + hard
        sd = sd + jnp.sum((x0 - qsoft) ** 2)
        hd = hd + jnp.sum((x0 - qhard) ** 2)
        codes_cols.append(code)

    codes_ref[...] = jnp.concatenate(codes_cols, axis=1)

    jc = jnp.sum((qsoft - qhard) ** 2)
    contrib = (0.1 * sd + hd + 0.1 * jc) * (1.0 / (_N * _D))
    lane0 = (jax.lax.broadcasted_iota(jnp.int32, (1, 128), 1) == 0)
    row = jnp.where(lane0, contrib, 0.0)

    @pl.when(pid == 0)
    def _init():
        loss_ref[...] = jnp.zeros_like(loss_ref)

    loss_ref[...] += row


def kernel(x, codebook, scale):
    nblk = _N // _BLK
    codes, loss_row = pl.pallas_call(
        _drq_block_kernel,
        grid=(nblk,),
        in_specs=[
            pl.BlockSpec(memory_space=pltpu.SMEM),
            pl.BlockSpec((_BLK, _D), lambda i: (i, 0)),
            pl.BlockSpec((_K, _D), lambda i: (0, 0)),
            pl.BlockSpec((_D, _K), lambda i: (0, 0)),
        ],
        out_specs=[
            pl.BlockSpec((_BLK, _M), lambda i: (i, 0)),
            pl.BlockSpec((1, 128), lambda i: (0, 0)),
        ],
        out_shape=[
            jax.ShapeDtypeStruct((_N, _M), jnp.int32),
            jax.ShapeDtypeStruct((1, 128), jnp.float32),
        ],
        compiler_params=pltpu.CompilerParams(
            dimension_semantics=("arbitrary",)),
    )(scale, x, codebook, codebook.T)
    return (codes, loss_row[0, 0])


# speedup vs baseline: 3.1621x; 1.5238x over previous
import jax
import jax.numpy as jnp
from jax.experimental import pallas as pl
from jax.experimental.pallas import tpu as pltpu

_M = 4
_K = 1024
_D = 64
_N = 16384
_BLK = 512


def _drq_block_kernel(scale_ref, x_ref, cbt_ref, aug_ref, codes_ref, part_ref):
    ct = cbt_ref[...]
    aug = aug_ref[...]
    x0 = x_ref[...]
    r = x0
    qsoft = jnp.zeros_like(x0)
    qhard = jnp.zeros_like(x0)
    lane = jax.lax.broadcasted_iota(jnp.int32, (1, 128), 1)

    sd = jnp.float32(0.0)
    hd = jnp.float32(0.0)
    codes_cols = []
    for i in range(_M):
        s = scale_ref[i]
        svec = jnp.where(lane < _D, s, 1.0)
        saug = aug * svec
        ci_sq = saug[:, :_D] * saug[:, :_D]
        cn = jnp.sum(ci_sq, axis=1)
        rn = jnp.sum(r * r, axis=1, keepdims=True)
        dots2 = jax.lax.dot_general(
            r + r, ct * s, (((1,), (0,)), ((), ())),
            preferred_element_type=jnp.float32)
        d = (rn - dots2) + cn[None, :]
        mn = jnp.min(d, axis=1, keepdims=True)
        e = jnp.exp(mn - d)
        mask = (d == mn).astype(jnp.float32)
        out1 = jax.lax.dot_general(
            e, saug, (((1,), (0,)), ((), ())),
            preferred_element_type=jnp.float32)
        out2 = jax.lax.dot_general(
            mask, saug, (((1,), (0,)), ((), ())),
            preferred_element_type=jnp.float32)
        soft = out1[:, :_D] / out1[:, _D:_D + 1]
        hard = out2[:, :_D]
        code_f = out2[:, _D + 1:_D + 2] * 256.0 + out2[:, _D + 2:_D + 3]
        r = r - hard
        qsoft = qsoft + soft
        qhard = qhard + hard
        sd = sd + jnp.sum((x0 - qsoft) ** 2)
        hd = hd + jnp.sum((x0 - qhard) ** 2)
        codes_cols.append(code_f.astype(jnp.int32))

    codes_ref[...] = jnp.concatenate(codes_cols, axis=1)

    jc = jnp.sum((qsoft - qhard) ** 2)
    contrib = (0.1 * sd + hd + 0.1 * jc) * (1.0 / (_N * _D))
    part_ref[...] = jnp.where(lane == 0, contrib, 0.0)[None]


def _loss_reduce_kernel(part_ref, out_ref):
    out_ref[...] = jnp.sum(part_ref[...], axis=0)


def kernel(x, codebook, scale):
    nblk = _N // _BLK
    ids = jnp.arange(_K, dtype=jnp.float32)
    aug = jnp.zeros((_K, 128), jnp.float32)
    aug = aug.at[:, :_D].set(codebook)
    aug = aug.at[:, _D].set(1.0)
    aug = aug.at[:, _D + 1].set(jnp.floor(ids / 256.0))
    aug = aug.at[:, _D + 2].set(ids - 256.0 * jnp.floor(ids / 256.0))

    codes, parts = pl.pallas_call(
        _drq_block_kernel,
        grid=(nblk,),
        in_specs=[
            pl.BlockSpec(memory_space=pltpu.SMEM),
            pl.BlockSpec((_BLK, _D), lambda i: (i, 0)),
            pl.BlockSpec((_D, _K), lambda i: (0, 0)),
            pl.BlockSpec((_K, 128), lambda i: (0, 0)),
        ],
        out_specs=[
            pl.BlockSpec((_BLK, _M), lambda i: (i, 0)),
            pl.BlockSpec((1, 1, 128), lambda i: (i, 0, 0)),
        ],
        out_shape=[
            jax.ShapeDtypeStruct((_N, _M), jnp.int32),
            jax.ShapeDtypeStruct((nblk, 1, 128), jnp.float32),
        ],
        compiler_params=pltpu.CompilerParams(
            dimension_semantics=("parallel",)),
    )(scale, x, codebook.T, aug)

    loss_row = pl.pallas_call(
        _loss_reduce_kernel,
        out_shape=jax.ShapeDtypeStruct((1, 128), jnp.float32),
    )(parts)
    return (codes, loss_row[0, 0])


# speedup vs baseline: 3.4033x; 1.0763x over previous
import jax
import jax.numpy as jnp
from jax.experimental import pallas as pl
from jax.experimental.pallas import tpu as pltpu

_M = 4
_K = 1024
_D = 64
_N = 16384
_BLK = 512


def _drq_block_kernel(x_ref, saug_ref, cts_ref, cn_ref, codes_ref, loss_ref):
    pid = pl.program_id(0)
    x0 = x_ref[...]
    r = x0
    qsoft = jnp.zeros_like(x0)
    qhard = jnp.zeros_like(x0)
    acc_sd = jnp.zeros_like(x0)
    acc_hd = jnp.zeros_like(x0)
    lane = jax.lax.broadcasted_iota(jnp.int32, (1, 128), 1)

    codes_cols = []
    for i in range(_M):
        saug = saug_ref[i]
        cn = cn_ref[i, :]
        rn = jnp.sum(r * r, axis=1, keepdims=True)
        dots2 = jax.lax.dot_general(
            r + r, cts_ref[i], (((1,), (0,)), ((), ())),
            preferred_element_type=jnp.float32)
        d = (rn - dots2) + cn[None, :]
        mn = jnp.min(d, axis=1, keepdims=True)
        e = jnp.exp(mn - d)
        mask = (d == mn).astype(jnp.float32)
        out1 = jax.lax.dot_general(
            e, saug, (((1,), (0,)), ((), ())),
            preferred_element_type=jnp.float32)
        out2 = jax.lax.dot_general(
            mask, saug, (((1,), (0,)), ((), ())),
            preferred_element_type=jnp.float32)
        soft = out1[:, :_D] / out1[:, _D:_D + 1]
        hard = out2[:, :_D]
        code_f = out2[:, _D + 1:_D + 2] * 256.0 + out2[:, _D + 2:_D + 3]
        r = r - hard
        qsoft = qsoft + soft
        qhard = qhard + hard
        dso = x0 - qsoft
        dha = x0 - qhard
        acc_sd = acc_sd + dso * dso
        acc_hd = acc_hd + dha * dha
        codes_cols.append(code_f.astype(jnp.int32))

    codes_ref[...] = jnp.concatenate(codes_cols, axis=1)

    djc = qsoft - qhard
    blk = 0.1 * jnp.sum(acc_sd) + jnp.sum(acc_hd) + 0.1 * jnp.sum(djc * djc)
    contrib = blk * (1.0 / (_N * _D))
    row = jnp.where(lane == 0, contrib, 0.0)

    @pl.when(pid == 0)
    def _init():
        loss_ref[...] = jnp.zeros_like(loss_ref)

    loss_ref[...] += row


def kernel(x, codebook, scale):
    nblk = _N // _BLK
    ids = jnp.arange(_K, dtype=jnp.float32)
    hi = jnp.floor(ids / 256.0)
    lo = ids - 256.0 * hi
    const_cols = jnp.concatenate(
        [jnp.ones((_K, 1), jnp.float32), hi[:, None], lo[:, None],
         jnp.zeros((_K, 128 - _D - 3), jnp.float32)], axis=1)
    ci_all = codebook[None, :, :] * scale[:, None, None]
    saug_all = jnp.concatenate(
        [ci_all, jnp.broadcast_to(const_cols[None], (_M, _K, 128 - _D))],
        axis=2)
    cts_all = jnp.transpose(ci_all, (0, 2, 1))
    cn_all = jnp.sum(ci_all * ci_all, axis=2)

    codes, loss_row = pl.pallas_call(
        _drq_block_kernel,
        grid=(nblk,),
        in_specs=[
            pl.BlockSpec((_BLK, _D), lambda i: (i, 0)),
            pl.BlockSpec((_M, _K, 128), lambda i: (0, 0, 0)),
            pl.BlockSpec((_M, _D, _K), lambda i: (0, 0, 0)),
            pl.BlockSpec((_M, _K), lambda i: (0, 0)),
        ],
        out_specs=[
            pl.BlockSpec((_BLK, _M), lambda i: (i, 0)),
            pl.BlockSpec((1, 128), lambda i: (0, 0)),
        ],
        out_shape=[
            jax.ShapeDtypeStruct((_N, _M), jnp.int32),
            jax.ShapeDtypeStruct((1, 128), jnp.float32),
        ],
        compiler_params=pltpu.CompilerParams(
            dimension_semantics=("arbitrary",)),
    )(x, saug_all, cts_all, cn_all)
    return (codes, loss_row[0, 0])
